# baseline (speedup 1.0000x reference)
import functools

import jax
import jax.numpy as jnp
from jax import lax
from jax.experimental import pallas as pl
from jax.experimental.pallas import tpu as pltpu
from jax.experimental.pallas import tpu_sc as plsc

NC, NS, L = 2, 16, 16
NW = NC * NS
B, SEQ, D = 4, 8192, 1024

SEQ_SC = 256
SEQ_TC = SEQ - SEQ_SC

SPW = SEQ_SC // NW
T = 4
TILES = SPW // T

_mesh = plsc.VectorSubcoreMesh(core_axis_name="c", subcore_axis_name="s")

_scratch = (
    [pltpu.VMEM((T, D), jnp.float32) for _ in range(8)]
    + [pltpu.VMEM((T, D), jnp.float32) for _ in range(2)]
    + [pltpu.SemaphoreType.DMA for _ in range(18)]
)


@functools.partial(
    pl.kernel,
    out_type=jax.ShapeDtypeStruct((B, SEQ_SC, D), jnp.float32),
    mesh=_mesh,
    scratch_types=_scratch,
)
def _sc_pos_add(x_hbm, pos_hbm, out_hbm, *refs):
    xb = [[refs[2 * b + p] for p in (0, 1)] for b in range(4)]
    pos_b = [refs[8], refs[9]]
    in_sem = [[refs[10 + 2 * b + p] for p in (0, 1)] for b in range(4)]
    out_sem = [[refs[18 + 2 * b + p] for p in (0, 1)] for b in range(4)]
    pos_sem = [refs[26], refs[27]]

    wid = lax.axis_index("s") * NC + lax.axis_index("c")
    seq0 = wid * SPW

    pltpu.async_copy(pos_hbm.at[pl.ds(SEQ_TC + seq0, T)], pos_b[0], pos_sem[0])
    for b in range(4):
        pltpu.async_copy(x_hbm.at[b, pl.ds(SEQ_TC + seq0, T)], xb[b][0],
                         in_sem[b][0])

    def tile_step(t, p):
        q = 1 - p
        tn = t + 1

        @pl.when(tn < TILES)
        def _():
            pltpu.async_copy(
                pos_hbm.at[pl.ds(SEQ_TC + seq0 + tn * T, T)], pos_b[q],
                pos_sem[q])

        pltpu.make_async_copy(pos_hbm.at[pl.ds(0, T)], pos_b[p],
                              pos_sem[p]).wait()

        for b in range(4):
            @pl.when(t > 0)
            def _():
                pltpu.make_async_copy(xb[b][q], out_hbm.at[b, pl.ds(0, T)],
                                      out_sem[b][q]).wait()

            @pl.when(tn < TILES)
            def _():
                pltpu.async_copy(
                    x_hbm.at[b, pl.ds(SEQ_TC + seq0 + tn * T, T)],
                    xb[b][q], in_sem[b][q])

            pltpu.make_async_copy(x_hbm.at[b, pl.ds(0, T)], xb[b][p],
                                  in_sem[b][p]).wait()

            def row_body(r, carry):
                def add_body(c, carry2):
                    sl = pl.ds(c * L, L)
                    plsc.addupdate(xb[b][p].at[r, sl], pos_b[p][r, sl])
                    return carry2

                return lax.fori_loop(0, D // L, add_body, carry, unroll=8)

            lax.fori_loop(0, T, row_body, 0)

            pltpu.async_copy(xb[b][p],
                             out_hbm.at[b, pl.ds(seq0 + t * T, T)],
                             out_sem[b][p])

    @pl.loop(0, TILES, step=2)
    def _(tp):
        tile_step(tp, 0)
        tile_step(tp + 1, 1)

    for b in range(4):
        pltpu.make_async_copy(xb[b][1], out_hbm.at[b, pl.ds(0, T)],
                              out_sem[b][1]).wait()


BS = 1984


def _tc_body(x_ref, pos_ref, out_ref):
    out_ref[...] = x_ref[...] + pos_ref[...][None]


def _tc_add(x, pos_table):
    return pl.pallas_call(
        _tc_body,
        grid=(SEQ_TC // BS, B),
        in_specs=[
            pl.BlockSpec((1, BS, D), lambda i, b: (b, i, 0)),
            pl.BlockSpec((BS, D), lambda i, b: (i, 0)),
        ],
        out_specs=pl.BlockSpec((1, BS, D), lambda i, b: (b, i, 0)),
        out_shape=jax.ShapeDtypeStruct(x.shape, x.dtype),
    )(x, pos_table)


def kernel(x, pos_table):
    y = _tc_add(x, pos_table)
    sc_out = _sc_pos_add(x, pos_table)
    return lax.dynamic_update_slice(y, sc_out, (0, SEQ_TC, 0))
